# baseline (speedup 1.0000x reference)
import jax
import jax.numpy as jnp
from jax.experimental import pallas as pl


_C_CHUNK = 32
_HT = 32


def _lroll(v, sh):
    n = v.shape[-1]
    sh = sh % n
    if sh == 0:
        return v
    ax = v.ndim - 1
    a = jax.lax.slice_in_dim(v, n - sh, n, axis=ax)
    b = jax.lax.slice_in_dim(v, 0, n - sh, axis=ax)
    return jax.lax.concatenate([a, b], ax)


def _make_fused_kernel(NS):
    def _fused_kernel(wt_ref, x_ref, ab_ref, be_ref, o_ref):
        xm = x_ref[0]
        above = ab_ref[0, :, 7:8, :]
        below = be_ref[0, :, 0:1, :]
        C, Ht, W = xm.shape
        H = Ht * NS
        h = pl.program_id(1)
        i0 = h * Ht

        xs = jnp.concatenate([xm, below], axis=1)
        xep = jnp.concatenate([above, xm], axis=1)

        zero = jnp.zeros((Ht + 1, W), jnp.float32)
        d_row, d_col, d_dia, d_adi = zero, zero, zero, zero
        for c0 in range(0, C, _C_CHUNK):
            a = xs[c0:c0 + _C_CHUNK]
            p = xep[c0:c0 + _C_CHUNK]
            d = a - _lroll(a, 1)
            d_row = d_row + jnp.sum(d * d, axis=0)
            d = a - p
            d_col = d_col + jnp.sum(d * d, axis=0)
            d = a - _lroll(p, 1)
            d_dia = d_dia + jnp.sum(d * d, axis=0)
            d = a - _lroll(p, -1)
            d_adi = d_adi + jnp.sum(d * d, axis=0)

        riota = jax.lax.broadcasted_iota(jnp.int32, (Ht + 1, W), 0)
        colx = jax.lax.broadcasted_iota(jnp.int32, (Ht + 1, W), 1)

        strip_l = xs[:, :, 0:1]
        strip_r = xep[:, :, W - 1:W]
        d = strip_l - strip_r
        fix_row = jnp.sum(d * d, axis=0)
        d_row = jnp.where(colx == 0, fix_row, d_row)

        first = h == 0
        last = h == NS - 1
        top0 = xs[:, 0:1, :]
        topT = xs[:, Ht:Ht + 1, :]
        abv0 = xep[:, 0:1, :]
        abvT = xep[:, Ht:Ht + 1, :]

        def fixrow0(t, ab, sh):
            d = t - _lroll(ab, sh)
            return jnp.sum(d * d, axis=0)

        m0 = first & (riota == 0)
        mT = last & (riota == Ht)
        d_col = jnp.where(m0, fixrow0(top0, abv0, 1), d_col)
        d_col = jnp.where(mT, fixrow0(topT, abvT, 1), d_col)
        d_dia = jnp.where(m0, fixrow0(top0, abv0, 2), d_dia)
        d_dia = jnp.where(mT, fixrow0(topT, abvT, 2), d_dia)
        d_adi = jnp.where(m0, fixrow0(top0, abv0, 0), d_adi)
        d_adi = jnp.where(mT, fixrow0(topT, abvT, 0), d_adi)

        rowm = riota[:Ht] + i0
        colm = colx[:Ht]
        wrapm = rowm == H - 1

        A = _lroll(d_row, -1)
        f2 = jnp.where(colm == W - 1, A[1:], A[:-1])
        f3 = jnp.where(wrapm, _lroll(d_col, -1)[1:], d_col[1:])
        f6 = jnp.where(wrapm, _lroll(d_dia, -2)[1:], _lroll(d_dia, -1)[1:])
        f7 = jnp.where(wrapm, d_adi[1:], _lroll(d_adi, 1)[1:])

        s_tl = jnp.sum(top0[:, :, 0:1] * top0[:, :, 0:1], axis=0)
        botm = xs[:, Ht - 1:Ht, :]
        s_bot = jnp.sum(botm * botm, axis=0)
        at00 = first & (rowm == 0) & (colm == 0)
        d_rowm = jnp.where(at00, s_tl, d_row[:Ht])
        d_colm = jnp.where(at00, s_tl, d_col[:Ht])
        d_diam = jnp.where(at00, s_tl, d_dia[:Ht])
        d_adim = jnp.where(at00, s_tl, d_adi[:Ht])
        f2 = jnp.where(wrapm & (colm == W - 1), s_bot, f2)
        f3 = jnp.where(wrapm & (colm == W - 1), s_bot, f3)
        f6 = jnp.where(wrapm & (colm == W - 2), s_bot, f6)
        f7 = jnp.where(wrapm & (colm == 0), s_bot, f7)

        def t(z):
            return jnp.tanh(jnp.sqrt(z + 1e-12))

        G = 1.6 + 0.8 * (t(d_rowm) + t(f2) + t(d_colm) + t(f3)
                         + t(d_diam) + t(f6) + t(d_adim) + t(f7))

        z2 = jnp.reshape(xm * G[None], (C, Ht * W))
        o2 = jnp.dot(wt_ref[...], z2, preferred_element_type=jnp.float32)
        o_ref[0] = xm + jnp.reshape(o2, (C, Ht, W))

    return _fused_kernel


def kernel(x, W_out):
    B, C, H, W = x.shape
    wt = W_out.T
    Ht = _HT
    NS = H // Ht
    nh8 = H // 8

    out = pl.pallas_call(
        _make_fused_kernel(NS),
        grid=(B, NS),
        in_specs=[
            pl.BlockSpec((C, C), lambda b, h: (0, 0)),
            pl.BlockSpec((1, C, Ht, W), lambda b, h: (b, 0, h, 0)),
            pl.BlockSpec((1, C, 8, W),
                         lambda b, h: (b, 0, (h * Ht // 8 - 1) % nh8, 0)),
            pl.BlockSpec((1, C, 8, W),
                         lambda b, h: (b, 0, ((h + 1) * Ht // 8) % nh8, 0)),
        ],
        out_specs=pl.BlockSpec((1, C, Ht, W), lambda b, h: (b, 0, h, 0)),
        out_shape=jax.ShapeDtypeStruct((B, C, H, W), jnp.float32),
    )(wt, x, x, x)

    return out


# speedup vs baseline: 1.1594x; 1.1594x over previous
import jax
import jax.numpy as jnp
from jax.experimental import pallas as pl


_C_CHUNK = 32


def _roll(v, sh, axis):
    n = v.shape[axis]
    sh = sh % n
    if sh == 0:
        return v
    axis = axis % v.ndim
    a = jax.lax.slice_in_dim(v, n - sh, n, axis=axis)
    b = jax.lax.slice_in_dim(v, 0, n - sh, axis=axis)
    return jax.lax.concatenate([a, b], axis)


def _roll2(v, di, dj):
    return _roll(_roll(v, di, -2), dj, -1)


def _fused_kernel(wt_ref, x_ref, o_ref):
    xb = x_ref[0]
    C, H, W = xb.shape
    row = jax.lax.broadcasted_iota(jnp.int32, (H, W), 0)
    col = jax.lax.broadcasted_iota(jnp.int32, (H, W), 1)

    zero = jnp.zeros((H, W), jnp.float32)
    d_row, d_col, d_dia, d_adi = zero, zero, zero, zero
    for c0 in range(0, C, _C_CHUNK):
        xc = xb[c0:c0 + _C_CHUNK]
        xu = _roll(xc, 1, -2)
        d = xc - _roll(xc, 1, -1)
        d_row = d_row + jnp.sum(d * d, axis=0)
        d = xc - xu
        d_col = d_col + jnp.sum(d * d, axis=0)
        d = xc - _roll(xu, 1, -1)
        d_dia = d_dia + jnp.sum(d * d, axis=0)
        d = xc - _roll(xu, -1, -1)
        d_adi = d_adi + jnp.sum(d * d, axis=0)

    left = xb[:, :, 0:1]
    rightr = _roll(xb[:, :, W - 1:W], 1, 1)
    top = xb[:, 0:1, :]
    bot = xb[:, H - 1:H, :]

    d = left - rightr
    fix_row = jnp.sum(d * d, axis=0)
    d = top - _roll(bot, 1, 2)
    fix_col = jnp.sum(d * d, axis=0)
    d = top - _roll(bot, 2, 2)
    fix_dia = jnp.sum(d * d, axis=0)
    d = top - bot
    fix_adi = jnp.sum(d * d, axis=0)

    d_row = jnp.where(col == 0, fix_row, d_row)
    d_col = jnp.where(row == 0, fix_col, d_col)
    d_dia = jnp.where(row == 0, fix_dia, d_dia)
    d_adi = jnp.where(row == 0, fix_adi, d_adi)

    f2 = jnp.where(col == W - 1, _roll2(d_row, -1, -1), _roll2(d_row, 0, -1))
    f3 = jnp.where(row == H - 1, _roll2(d_col, -1, -1), _roll2(d_col, -1, 0))
    f6 = jnp.where(row == H - 1, _roll2(d_dia, -1, -2), _roll2(d_dia, -1, -1))
    f7 = jnp.where(row == H - 1, _roll2(d_adi, -1, 0), _roll2(d_adi, -1, 1))

    s_tl = jnp.sum(top[:, :, 0:1] * top[:, :, 0:1], axis=0)
    s_bot = jnp.sum(bot * bot, axis=0)
    at00 = (row == 0) & (col == 0)
    mbot = row == H - 1
    d_row = jnp.where(at00, s_tl, d_row)
    d_col = jnp.where(at00, s_tl, d_col)
    d_dia = jnp.where(at00, s_tl, d_dia)
    d_adi = jnp.where(at00, s_tl, d_adi)
    f2 = jnp.where(mbot & (col == W - 1), s_bot, f2)
    f3 = jnp.where(mbot & (col == W - 1), s_bot, f3)
    f6 = jnp.where(mbot & (col == W - 2), s_bot, f6)
    f7 = jnp.where(mbot & (col == 0), s_bot, f7)

    def t(z):
        return jnp.tanh(jnp.sqrt(z + 1e-12))

    G = 1.6 + 0.8 * (t(d_row) + t(f2) + t(d_col) + t(f3)
                     + t(d_dia) + t(f6) + t(d_adi) + t(f7))

    z2 = jnp.reshape((xb * G[None]).astype(jnp.bfloat16), (C, H * W))
    o2 = jnp.dot(wt_ref[...], z2, preferred_element_type=jnp.float32)
    o_ref[0] = xb + jnp.reshape(o2, (C, H, W))


def kernel(x, W_out):
    B, C, H, W = x.shape
    wt = W_out.T.astype(jnp.bfloat16)

    out = pl.pallas_call(
        _fused_kernel,
        grid=(B,),
        in_specs=[
            pl.BlockSpec((C, C), lambda b: (0, 0)),
            pl.BlockSpec((1, C, H, W), lambda b: (b, 0, 0, 0)),
        ],
        out_specs=pl.BlockSpec((1, C, H, W), lambda b: (b, 0, 0, 0)),
        out_shape=jax.ShapeDtypeStruct((B, C, H, W), jnp.float32),
    )(wt, x)

    return out


# speedup vs baseline: 1.4038x; 1.2107x over previous
import jax
import jax.numpy as jnp
from jax.experimental import pallas as pl


def _roll(v, sh, axis):
    n = v.shape[axis]
    sh = sh % n
    if sh == 0:
        return v
    axis = axis % v.ndim
    a = jax.lax.slice_in_dim(v, n - sh, n, axis=axis)
    b = jax.lax.slice_in_dim(v, 0, n - sh, axis=axis)
    return jax.lax.concatenate([a, b], axis)


def _roll2(v, di, dj):
    return _roll(_roll(v, di, -2), dj, -1)


def _fused_kernel(wt_ref, x_ref, o_ref):
    xb = x_ref[0]
    C, H, W = xb.shape
    L = H * W
    row = jax.lax.broadcasted_iota(jnp.int32, (H, W), 0)
    col = jax.lax.broadcasted_iota(jnp.int32, (H, W), 1)

    xh = xb.astype(jnp.bfloat16)
    x2 = jnp.reshape(xh, (C, L))
    r1 = _roll(x2, 1, -1)
    rW = _roll(x2, W, -1)
    rW1 = _roll(r1, W, -1)
    rWm = _roll(rW, -1, -1)

    ones = jnp.ones((8, C), jnp.bfloat16)

    def field(p):
        d = x2 - p
        e = jnp.dot(ones, d * d, preferred_element_type=jnp.float32)
        return jnp.reshape(e[0:1], (H, W))

    d_row = field(r1)
    d_col = field(rW)
    d_dia = field(rW1)
    d_adi = field(rWm)

    top = xb[:, 0:1, :]
    bot = xb[:, H - 1:H, :]
    rgt = xb[:, :, W - 1:W]
    lftr = _roll(xb[:, :, 0:1], 1, 1)

    d = top - _roll(bot, 1, 2)
    fix_col = jnp.sum(d * d, axis=0)
    d = top - _roll(bot, 2, 2)
    fix_dia = jnp.sum(d * d, axis=0)
    d = top - bot
    fix_adi = jnp.sum(d * d, axis=0)
    d = rgt - lftr
    fix_adc = jnp.sum(d * d, axis=0)

    d_dia = jnp.where(col == 0, d_row, d_dia)
    d_adi = jnp.where(col == W - 1, fix_adc, d_adi)
    d_col = jnp.where(row == 0, fix_col, d_col)
    d_dia = jnp.where(row == 0, fix_dia, d_dia)
    d_adi = jnp.where(row == 0, fix_adi, d_adi)

    f2 = jnp.where(col == W - 1, _roll2(d_row, -1, -1), _roll2(d_row, 0, -1))
    f3 = jnp.where(row == H - 1, _roll2(d_col, -1, -1), _roll2(d_col, -1, 0))
    f6 = jnp.where(row == H - 1, _roll2(d_dia, -1, -2), _roll2(d_dia, -1, -1))
    f7 = jnp.where(row == H - 1, _roll2(d_adi, -1, 0), _roll2(d_adi, -1, 1))

    s_tl = jnp.sum(top[:, :, 0:1] * top[:, :, 0:1], axis=0)
    s_bot = jnp.sum(bot * bot, axis=0)
    at00 = (row == 0) & (col == 0)
    mbot = row == H - 1
    d_row = jnp.where(at00, s_tl, d_row)
    d_col = jnp.where(at00, s_tl, d_col)
    d_dia = jnp.where(at00, s_tl, d_dia)
    d_adi = jnp.where(at00, s_tl, d_adi)
    f2 = jnp.where(mbot & (col == W - 1), s_bot, f2)
    f3 = jnp.where(mbot & (col == W - 1), s_bot, f3)
    f6 = jnp.where(mbot & (col == W - 2), s_bot, f6)
    f7 = jnp.where(mbot & (col == 0), s_bot, f7)

    def t(z):
        return jnp.tanh(jnp.sqrt(z + 1e-12))

    G = 1.6 + 0.8 * (t(d_row) + t(f2) + t(d_col) + t(f3)
                     + t(d_dia) + t(f6) + t(d_adi) + t(f7))

    g2 = jnp.reshape(G.astype(jnp.bfloat16), (1, L))
    o2 = jnp.dot(wt_ref[...], x2 * g2, preferred_element_type=jnp.float32)
    o_ref[0] = xb + jnp.reshape(o2, (C, H, W))


def kernel(x, W_out):
    B, C, H, W = x.shape
    wt = W_out.T.astype(jnp.bfloat16)

    out = pl.pallas_call(
        _fused_kernel,
        grid=(B,),
        in_specs=[
            pl.BlockSpec((C, C), lambda b: (0, 0)),
            pl.BlockSpec((1, C, H, W), lambda b: (b, 0, 0, 0)),
        ],
        out_specs=pl.BlockSpec((1, C, H, W), lambda b: (b, 0, 0, 0)),
        out_shape=jax.ShapeDtypeStruct((B, C, H, W), jnp.float32),
    )(wt, x)

    return out


# speedup vs baseline: 1.5041x; 1.0715x over previous
import jax
import jax.numpy as jnp
from jax.experimental import pallas as pl


def _roll(v, sh, axis):
    n = v.shape[axis]
    sh = sh % n
    if sh == 0:
        return v
    axis = axis % v.ndim
    a = jax.lax.slice_in_dim(v, n - sh, n, axis=axis)
    b = jax.lax.slice_in_dim(v, 0, n - sh, axis=axis)
    return jax.lax.concatenate([a, b], axis)


def _roll2(v, di, dj):
    return _roll(_roll(v, di, -2), dj, -1)


def _fused_kernel(wt_ref, x_ref, o_ref):
    xb = x_ref[0]
    C, H, W = xb.shape
    L = H * W
    row = jax.lax.broadcasted_iota(jnp.int32, (H, W), 0)
    col = jax.lax.broadcasted_iota(jnp.int32, (H, W), 1)

    xh = xb.astype(jnp.bfloat16)
    x2 = jnp.reshape(xh, (C, L))
    r1 = _roll(x2, 1, -1)
    rW = _roll(x2, W, -1)
    rW1 = _roll(r1, W, -1)
    rWm = _roll(rW, -1, -1)

    ones = jnp.ones((8, C), jnp.bfloat16)

    def field(p):
        d = x2 - p
        e = jnp.dot(ones, d * d, preferred_element_type=jnp.float32)
        return jnp.reshape(e[0:1], (H, W))

    d_row = field(r1)
    d_col = field(rW)
    d_dia = field(rW1)
    d_adi = field(rWm)

    top = xb[:, 0:1, :]
    bot = xb[:, H - 1:H, :]
    rgt = xb[:, :, W - 1:W]
    lftr = _roll(xb[:, :, 0:1], 1, 1)

    d = top - _roll(bot, 1, 2)
    fix_col = jnp.sum(d * d, axis=0)
    d = top - _roll(bot, 2, 2)
    fix_dia = jnp.sum(d * d, axis=0)
    d = top - bot
    fix_adi = jnp.sum(d * d, axis=0)
    d = rgt - lftr
    fix_adc = jnp.sum(d * d, axis=0)

    d_dia = jnp.where(col == 0, d_row, d_dia)
    d_adi = jnp.where(col == W - 1, fix_adc, d_adi)
    d_col = jnp.where(row == 0, fix_col, d_col)
    d_dia = jnp.where(row == 0, fix_dia, d_dia)
    d_adi = jnp.where(row == 0, fix_adi, d_adi)

    f2 = jnp.where(col == W - 1, _roll2(d_row, -1, -1), _roll2(d_row, 0, -1))
    f3 = jnp.where(row == H - 1, _roll2(d_col, -1, -1), _roll2(d_col, -1, 0))
    f6 = jnp.where(row == H - 1, _roll2(d_dia, -1, -2), _roll2(d_dia, -1, -1))
    f7 = jnp.where(row == H - 1, _roll2(d_adi, -1, 0), _roll2(d_adi, -1, 1))

    s_tl = jnp.sum(top[:, :, 0:1] * top[:, :, 0:1], axis=0)
    s_bot = jnp.sum(bot * bot, axis=0)
    at00 = (row == 0) & (col == 0)
    mbot = row == H - 1
    d_row = jnp.where(at00, s_tl, d_row)
    d_col = jnp.where(at00, s_tl, d_col)
    d_dia = jnp.where(at00, s_tl, d_dia)
    d_adi = jnp.where(at00, s_tl, d_adi)
    f2 = jnp.where(mbot & (col == W - 1), s_bot, f2)
    f3 = jnp.where(mbot & (col == W - 1), s_bot, f3)
    f6 = jnp.where(mbot & (col == W - 2), s_bot, f6)
    f7 = jnp.where(mbot & (col == 0), s_bot, f7)

    def t(z):
        return jnp.tanh(jnp.sqrt(z + 1e-12))

    G = 1.6 + 0.8 * (t(d_row) + t(f2) + t(d_col) + t(f3)
                     + t(d_dia) + t(f6) + t(d_adi) + t(f7))

    g2 = jnp.reshape(G.astype(jnp.bfloat16), (1, L))
    o2 = jnp.dot(wt_ref[...], x2 * g2,
                 preferred_element_type=jnp.float32).astype(jnp.bfloat16)
    o_ref[0] = xb + jnp.reshape(o2, (C, H, W)).astype(jnp.float32)


def kernel(x, W_out):
    B, C, H, W = x.shape
    wt = W_out.T.astype(jnp.bfloat16)

    out = pl.pallas_call(
        _fused_kernel,
        grid=(B,),
        in_specs=[
            pl.BlockSpec((C, C), lambda b: (0, 0)),
            pl.BlockSpec((1, C, H, W), lambda b: (b, 0, 0, 0)),
        ],
        out_specs=pl.BlockSpec((1, C, H, W), lambda b: (b, 0, 0, 0)),
        out_shape=jax.ShapeDtypeStruct((B, C, H, W), jnp.float32),
    )(wt, x)

    return out
